# baseline (speedup 1.0000x reference)
import jax
import jax.numpy as jnp
from jax.experimental import pallas as pl
from jax.experimental.pallas import tpu as pltpu

_BM = 200
_S = 2


def _gcn_body(x_ref, w_ref, *rest):
    adj_refs = rest[:_S]
    bias_ref = rest[_S]
    out_ref = rest[_S + 1]
    sup_ref = rest[_S + 2]

    @pl.when(pl.program_id(0) == 0)
    def _():
        sup_ref[...] = jnp.dot(
            x_ref[...].astype(jnp.bfloat16),
            w_ref[...].astype(jnp.bfloat16),
            preferred_element_type=jnp.float32,
        ).astype(jnp.bfloat16)

    for j in range(_S):
        out_ref[j * _BM:(j + 1) * _BM, :] = (
            jnp.dot(
                adj_refs[j][...].astype(jnp.bfloat16),
                sup_ref[...],
                preferred_element_type=jnp.float32,
            )
            + bias_ref[...]
        )


def kernel(x, adj, weight, bias):
    n, d_in = x.shape
    d_out = weight.shape[1]
    bm, s = _BM, _S
    rows_per_step = s * bm
    adj_specs = [
        pl.BlockSpec((bm, n), lambda i, j=j: (i * s + j, 0)) for j in range(s)
    ]
    return pl.pallas_call(
        _gcn_body,
        grid=(n // rows_per_step,),
        in_specs=[
            pl.BlockSpec((n, d_in), lambda i: (0, 0)),
            pl.BlockSpec((d_in, d_out), lambda i: (0, 0)),
            *adj_specs,
            pl.BlockSpec((1, d_out), lambda i: (0, 0)),
        ],
        out_specs=pl.BlockSpec((rows_per_step, d_out), lambda i: (i, 0)),
        out_shape=jax.ShapeDtypeStruct((n, d_out), x.dtype),
        scratch_shapes=[pltpu.VMEM((n, d_out), jnp.bfloat16)],
        compiler_params=pltpu.CompilerParams(
            dimension_semantics=("arbitrary",)
        ),
    )(x, weight, *([adj] * s), bias.reshape(1, d_out))
